# baseline (device time: 24632 ns/iter reference)
import jax
import jax.numpy as jnp
from jax import lax
from jax.experimental import pallas as pl
from jax.experimental.pallas import tpu as pltpu

N_DEV = 4
N_EXP = 8
EXP_PER = N_EXP // N_DEV


def kernel(x, router_W, route_idx, expert_W):
    m, d = x.shape
    _, _, h = expert_W.shape
    m_per = m // N_DEV

    def body(x_ref, rw_ref, idx_ref, ew_ref, out_ref,
             p_ref, comm_ref, send_sems, recv_sems):
        my = lax.axis_index("i")
        right = lax.rem(my + 1, N_DEV)

        xv = x_ref[:, :]
        scores = jnp.dot(xv, rw_ref[:, :], preferred_element_type=jnp.float32)
        smax = jnp.max(scores, axis=1, keepdims=True)
        probs = jnp.exp(scores - smax)
        probs = probs / jnp.sum(probs, axis=1, keepdims=True)

        idx0 = idx_ref[:, 0:1]
        idx1 = idx_ref[:, 1:2]
        eids = lax.broadcasted_iota(jnp.int32, (m, N_EXP), 1)
        top2 = (idx0 == eids) | (idx1 == eids)
        gsel = jnp.where(top2, probs, 0.0)
        g = gsel / jnp.sum(gsel, axis=1, keepdims=True)

        e_base = my * EXP_PER
        g0 = jnp.sum(jnp.where(eids == e_base, g, 0.0), axis=1, keepdims=True)
        g1 = jnp.sum(jnp.where(eids == e_base + 1, g, 0.0), axis=1, keepdims=True)

        p_ref[:, :] = (
            jnp.dot(g0 * xv, ew_ref[0], preferred_element_type=jnp.float32)
            + jnp.dot(g1 * xv, ew_ref[1], preferred_element_type=jnp.float32)
        )

        c0 = lax.rem(my + N_DEV - 1, N_DEV)
        comm_ref[0, :, :] = p_ref[pl.ds(c0 * m_per, m_per), :]
        for s in range(N_DEV - 1):
            rdma = pltpu.make_async_remote_copy(
                src_ref=comm_ref.at[s],
                dst_ref=comm_ref.at[s + 1],
                send_sem=send_sems.at[s],
                recv_sem=recv_sems.at[s],
                device_id=(right,),
                device_id_type=pl.DeviceIdType.MESH,
            )
            rdma.start()
            rdma.wait()
            c = lax.rem(my + 2 * N_DEV - s - 2, N_DEV)
            comm_ref[s + 1, :, :] = (
                comm_ref[s + 1, :, :] + p_ref[pl.ds(c * m_per, m_per), :]
            )

        out_ref[:, :] = comm_ref[N_DEV - 1, :, :]

    return pl.pallas_call(
        body,
        out_shape=jax.ShapeDtypeStruct((m_per, h), jnp.float32),
        in_specs=[
            pl.BlockSpec(memory_space=pltpu.VMEM),
            pl.BlockSpec(memory_space=pltpu.VMEM),
            pl.BlockSpec(memory_space=pltpu.VMEM),
            pl.BlockSpec(memory_space=pltpu.VMEM),
        ],
        out_specs=pl.BlockSpec(memory_space=pltpu.VMEM),
        scratch_shapes=[
            pltpu.VMEM((m, h), jnp.float32),
            pltpu.VMEM((N_DEV, m_per, h), jnp.float32),
            pltpu.SemaphoreType.DMA((N_DEV - 1,)),
            pltpu.SemaphoreType.DMA((N_DEV - 1,)),
        ],
    )(x, router_W, route_idx, expert_W)


# device time: 18412 ns/iter; 1.3378x vs baseline; 1.3378x over previous
import jax
import jax.numpy as jnp
from jax import lax
from jax.experimental import pallas as pl
from jax.experimental.pallas import tpu as pltpu

N_DEV = 4
N_EXP = 8
EXP_PER = N_EXP // N_DEV


def kernel(x, router_W, route_idx, expert_W):
    m, d = x.shape
    _, _, h = expert_W.shape
    m_per = m // N_DEV

    def body(x_ref, rw_ref, idx_ref, ew_ref, out_ref,
             y_ref, send_buf, recv_buf, send_sems, recv_sems):
        my = lax.axis_index("i")

        xv = x_ref[:, :]
        scores = jnp.dot(xv, rw_ref[:, :], preferred_element_type=jnp.float32)
        smax = jnp.max(scores, axis=1, keepdims=True)
        probs = jnp.exp(scores - smax)
        probs = probs / jnp.sum(probs, axis=1, keepdims=True)

        idx0 = idx_ref[:, 0:1]
        idx1 = idx_ref[:, 1:2]
        eids = lax.broadcasted_iota(jnp.int32, (m, N_EXP), 1)
        top2 = (idx0 == eids) | (idx1 == eids)
        gsel = jnp.where(top2, probs, 0.0)
        g = gsel / jnp.sum(gsel, axis=1, keepdims=True)

        e_base = my * EXP_PER
        g0 = jnp.sum(jnp.where(eids == e_base, g, 0.0), axis=1, keepdims=True)
        g1 = jnp.sum(jnp.where(eids == e_base + 1, g, 0.0), axis=1, keepdims=True)

        y_ref[0, :, :] = g0 * xv
        y_ref[1, :, :] = g1 * xv
        w0 = ew_ref[0]
        w1 = ew_ref[1]

        def chunk(dest):
            rows = pl.ds(dest * m_per, m_per)
            return (
                jnp.dot(y_ref[0, rows, :], w0, preferred_element_type=jnp.float32)
                + jnp.dot(y_ref[1, rows, :], w1, preferred_element_type=jnp.float32)
            )

        rdmas = []
        for t in range(1, N_DEV):
            s = t - 1
            dest = lax.rem(my + t, N_DEV)
            send_buf[s, :, :] = chunk(dest)
            rdma = pltpu.make_async_remote_copy(
                src_ref=send_buf.at[s],
                dst_ref=recv_buf.at[s],
                send_sem=send_sems.at[s],
                recv_sem=recv_sems.at[s],
                device_id=(dest,),
                device_id_type=pl.DeviceIdType.MESH,
            )
            rdma.start()
            rdmas.append(rdma)

        out_ref[:, :] = chunk(my)
        for s in range(N_DEV - 1):
            rdmas[s].wait_recv()
            out_ref[:, :] = out_ref[:, :] + recv_buf[s, :, :]
        for s in range(N_DEV - 1):
            rdmas[s].wait_send()

    return pl.pallas_call(
        body,
        out_shape=jax.ShapeDtypeStruct((m_per, h), jnp.float32),
        in_specs=[
            pl.BlockSpec(memory_space=pltpu.VMEM),
            pl.BlockSpec(memory_space=pltpu.VMEM),
            pl.BlockSpec(memory_space=pltpu.VMEM),
            pl.BlockSpec(memory_space=pltpu.VMEM),
        ],
        out_specs=pl.BlockSpec(memory_space=pltpu.VMEM),
        scratch_shapes=[
            pltpu.VMEM((EXP_PER, m, d), jnp.float32),
            pltpu.VMEM((N_DEV - 1, m_per, h), jnp.float32),
            pltpu.VMEM((N_DEV - 1, m_per, h), jnp.float32),
            pltpu.SemaphoreType.DMA((N_DEV - 1,)),
            pltpu.SemaphoreType.DMA((N_DEV - 1,)),
        ],
    )(x, router_W, route_idx, expert_W)
